# baseline (speedup 1.0000x reference)
import jax
import jax.numpy as jnp
from jax.experimental import pallas as pl
from jax.experimental.pallas import tpu as pltpu

_MAX_SIZE = 50
_HALF = 128
_BATCH = 64
_NSEM = 8


def _pos_broadcast_kernel(row_ref, col_ref, out_ref, scratch, sems):
    col = col_ref[...]
    for i in range(_MAX_SIZE):
        scratch[pl.ds(i * _MAX_SIZE, _MAX_SIZE), :_HALF] = jnp.broadcast_to(
            row_ref[pl.ds(i, 1), :], (_MAX_SIZE, _HALF)
        )
        scratch[pl.ds(i * _MAX_SIZE, _MAX_SIZE), _HALF:] = col

    for b in range(_BATCH):
        pltpu.make_async_copy(
            scratch, out_ref.at[b], sems.at[b % _NSEM]
        ).start()
    for b in range(_BATCH):
        pltpu.make_async_copy(
            scratch, out_ref.at[b], sems.at[b % _NSEM]
        ).wait()


def kernel(batch_size, row_embed, col_embed):
    zero = (jnp.asarray(batch_size) - _BATCH).astype(row_embed.dtype)
    row_embed = row_embed + zero
    col_embed = col_embed + zero

    return pl.pallas_call(
        _pos_broadcast_kernel,
        in_specs=[
            pl.BlockSpec(memory_space=pltpu.MemorySpace.VMEM),
            pl.BlockSpec(memory_space=pltpu.MemorySpace.VMEM),
        ],
        out_specs=pl.BlockSpec(memory_space=pltpu.MemorySpace.HBM),
        out_shape=jax.ShapeDtypeStruct(
            (_BATCH, _MAX_SIZE * _MAX_SIZE, 2 * _HALF), row_embed.dtype
        ),
        scratch_shapes=[
            pltpu.VMEM((_MAX_SIZE * _MAX_SIZE, 2 * _HALF), row_embed.dtype),
            pltpu.SemaphoreType.DMA((_NSEM,)),
        ],
    )(row_embed, col_embed)
